# baseline (speedup 1.0000x reference)
import math

import jax
import jax.numpy as jnp
from jax import lax
from jax.experimental import pallas as pl
from jax.experimental.pallas import tpu as pltpu
from jax.experimental.pallas import tpu_sc as plsc

L = 200
B = 4096
DIM = 64
SCALE = math.sqrt(DIM)

_info = plsc.get_sparse_core_info()
NC, NS = _info.num_cores, _info.num_subcores
NW = NC * NS
CH = B // NW
NBUF = 4
ROUNDS = L // NBUF
DT = DIM // 8
OUT_ROWS = L * DT * (B // 128)


def _sc_body(idx_hbm, w_hbm, pe_hbm, out_hbm, pe_v, idx_v, rin, tout, *sems):
    gsem = sems[:NBUF]
    ssem = sems[NBUF:]
    wid = lax.axis_index("s") * NC + lax.axis_index("c")
    col = wid * CH
    pltpu.sync_copy(pe_hbm, pe_v)
    pltpu.sync_copy(idx_hbm.at[:, pl.ds(col, CH)], idx_v)

    lane = lax.iota(jnp.int32, 16)
    row_idx = [lane + bg * 16 for bg in range(CH // 16)]

    def fire_gather(l, b):
        pltpu.async_copy(w_hbm.at[idx_v.at[l]], rin.at[b], gsem[b])

    def wait_gather(l, b):
        pltpu.make_async_copy(w_hbm.at[idx_v.at[l]], rin.at[b], gsem[b]).wait()

    def fire_store(l, b):
        for dt in range(DT):
            pltpu.async_copy(
                tout.at[b, pl.ds(dt * 8, 8)],
                out_hbm.at[l * (DT * 32) + dt * 32 + wid],
                ssem[b],
            )

    def wait_store(l, b):
        for dt in range(DT):
            pltpu.make_async_copy(
                tout.at[b, pl.ds(dt * 8, 8)],
                out_hbm.at[l * (DT * 32) + dt * 32 + wid],
                ssem[b],
            ).wait()

    def tfma(l, b):
        @plsc.parallel_loop(0, DIM, step=1, unroll=2)
        def _(d):
            cols = lax.broadcast(d, (16,))
            pe_d = plsc.load_gather(pe_v, [lax.broadcast(l, (16,)), cols])
            for bg in range(CH // 16):
                v = plsc.load_gather(rin.at[b], [row_idx[bg], cols])
                tout[b, d, pl.ds(bg * 16, 16)] = v * SCALE + pe_d

    def step(l, b, first, fire_next):
        wait_gather(l, b)
        if not first:
            wait_store(l - NBUF, b)
        tfma(l, b)
        fire_store(l, b)
        if fire_next:
            fire_gather(l + NBUF, b)

    for b in range(NBUF):
        fire_gather(b, b)
    for b in range(NBUF):
        step(b, b, first=True, fire_next=True)

    def round_body(mc, _):
        for b in range(NBUF):
            step(mc * NBUF + b, b, first=False, fire_next=True)
        return 0

    lax.fori_loop(1, ROUNDS - 1, round_body, 0)

    last = (ROUNDS - 1) * NBUF
    for b in range(NBUF):
        step(last + b, b, first=False, fire_next=False)
    for b in range(NBUF):
        wait_store(last + b, b)


@jax.jit
def _embed(idx, W, pe_s):
    mesh = plsc.VectorSubcoreMesh(core_axis_name="c", subcore_axis_name="s")
    f = pl.kernel(
        _sc_body,
        out_type=jax.ShapeDtypeStruct((OUT_ROWS, 8, 128), jnp.float32),
        mesh=mesh,
        scratch_types=[
            pltpu.VMEM((L, DIM), jnp.float32),
            pltpu.VMEM((L, CH), jnp.int32),
            pltpu.VMEM((NBUF, CH, DIM), jnp.float32),
            pltpu.VMEM((NBUF, DIM, CH), jnp.float32),
        ]
        + [pltpu.SemaphoreType.DMA] * (2 * NBUF),
        compiler_params=pltpu.CompilerParams(
            use_tc_tiling_on_sc=False, needs_layout_passes=False
        ),
    )
    return f(idx, W, pe_s)


def kernel(source, W, pe):
    idx = source.reshape(L, B)
    pe_s = pe[:L, 0, :]
    out5 = _embed(idx, W, pe_s).reshape(L, DT, B // 128, 8, 128)
    return out5.transpose(0, 2, 4, 1, 3).reshape(L, B, DIM)
